# Initial kernel scaffold; baseline (speedup 1.0000x reference)
import jax
import jax.numpy as jnp
from jax.experimental import pallas as pl


def kernel(x, h, edge_index, params):
    raise NotImplementedError("write your pallas kernel here")


# speedup vs baseline: 2.4049x; 2.4049x over previous
import functools

import jax
import jax.numpy as jnp
from jax import lax
from jax.experimental import pallas as pl
from jax.experimental.pallas import tpu as pltpu
from jax.experimental.pallas import tpu_sc as plsc

F32 = jnp.float32
_PREC = lax.Precision.HIGHEST

_N = 10000
_E = 320000
_D = 128
_C = 128
_NCH = _E // _C
_NC = 2
_NS = 16
_NW = _NC * _NS
_L = 16
_NP = 10240
_RPT = _NP // _NS
_RC = 128


def _silu(v):
    return v * jax.nn.sigmoid(v)


def _dot(a, b):
    return jnp.dot(a, b, preferred_element_type=F32, precision=_PREC)


def _sc_gather_body(a_hbm, b_hbm, x_hbm, eidx_hbm, wd_hbm, g_hbm,
                    x_v, wd_v, di_v, si_v, a_v, b_v, g_v, dist_v,
                    sem_a, sem_b):
    wid = lax.axis_index("s") * _NC + lax.axis_index("c")
    pltpu.sync_copy(x_hbm, x_v)
    pltpu.sync_copy(wd_hbm, wd_v)

    n_i = (_NCH - wid + _NW - 1) // _NW

    def chunk_body(i, carry):
        ci = wid + i * _NW
        pltpu.sync_copy(eidx_hbm.at[1, ci], di_v)
        pltpu.sync_copy(eidx_hbm.at[0, ci], si_v)
        cp_a = pltpu.async_copy(a_hbm.at[di_v], a_v, sem_a)
        cp_b = pltpu.async_copy(b_hbm.at[si_v], b_v, sem_b)
        for gi in range(_C // _L):
            sl = pl.ds(gi * _L, _L)
            d16 = di_v[sl]
            s16 = si_v[sl]
            d4 = d16 * 4
            s4 = s16 * 4
            acc = jnp.zeros((_L,), F32)
            for c in range(3):
                xd = plsc.load_gather(x_v, [d4 + c])
                xs = plsc.load_gather(x_v, [s4 + c])
                df = xd - xs
                acc = acc + df * df
            dist_v[sl] = acc
        cp_a.wait()
        cp_b.wait()

        def group_body(gi, carry2):
            dvec = dist_v[pl.ds(gi * _L, _L)]
            for jj in range(_L):
                j = gi * _L + jj
                d = dvec[jj]
                for k in range(_D // _L):
                    sl = pl.ds(k * _L, _L)
                    g_v[j, sl] = a_v[j, sl] + b_v[j, sl] + d * wd_v[sl]
            return carry2

        lax.fori_loop(0, _C // _L, group_body, 0, unroll=False)
        pltpu.sync_copy(g_v, g_hbm.at[pl.ds(ci * _C, _C), :])
        return carry

    lax.fori_loop(0, n_i, chunk_body, 0, unroll=False)


_sc_gather = functools.partial(
    pl.kernel,
    out_type=jax.ShapeDtypeStruct((_E, _D), F32),
    mesh=plsc.VectorSubcoreMesh(core_axis_name="c", subcore_axis_name="s"),
    compiler_params=pltpu.CompilerParams(needs_layout_passes=False),
    scratch_types=[
        pltpu.VMEM((_N * 4,), F32),
        pltpu.VMEM((_D,), F32),
        pltpu.VMEM((_C,), jnp.int32),
        pltpu.VMEM((_C,), jnp.int32),
        pltpu.VMEM((_C, _D), F32),
        pltpu.VMEM((_C, _D), F32),
        pltpu.VMEM((_C, _D), F32),
        pltpu.VMEM((_C,), F32),
        pltpu.SemaphoreType.DMA,
        pltpu.SemaphoreType.DMA,
    ],
)(_sc_gather_body)


def _sc_scatter_body(m2_hbm, eidx_hbm, out_hbm,
                     z_v, m_v, idx_v, acc_sh):
    cid = lax.axis_index("c")
    sid = lax.axis_index("s")

    def zrow(r, carry):
        for k in range(_D // _L):
            z_v[r, pl.ds(k * _L, _L)] = jnp.zeros((_L,), F32)
        return carry

    lax.fori_loop(0, _RC, zrow, 0, unroll=False)
    base = sid * _RPT
    for t in range(_RPT // _RC):
        pltpu.sync_copy(z_v, acc_sh.at[pl.ds(base + t * _RC, _RC), :])
    plsc.subcore_barrier()

    half = _NCH // _NC
    n_i = (half - sid + _NS - 1) // _NS

    def chunk_body(i, carry):
        ci = cid * half + sid + i * _NS
        pltpu.sync_copy(eidx_hbm.at[1, ci], idx_v.at[0])
        pltpu.sync_copy(m2_hbm.at[pl.ds(ci * _C, _C), :], m_v)
        pltpu.sync_copy(m_v, acc_sh.at[idx_v.at[0]], add=True)
        return carry

    lax.fori_loop(0, n_i, chunk_body, 0, unroll=False)
    plsc.subcore_barrier()
    for t in range(_RPT // _RC):
        sl = pl.ds(base + t * _RC, _RC)
        pltpu.sync_copy(acc_sh.at[sl, :], out_hbm.at[cid, sl, :])


_sc_scatter = functools.partial(
    pl.kernel,
    out_type=jax.ShapeDtypeStruct((_NC, _NP, _D), F32),
    mesh=plsc.VectorSubcoreMesh(core_axis_name="c", subcore_axis_name="s"),
    compiler_params=pltpu.CompilerParams(needs_layout_passes=False),
    scratch_types=[
        pltpu.VMEM((_RC, _D), F32),
        pltpu.VMEM((_C, _D), F32),
        pltpu.VMEM((1, _C), jnp.int32),
        pltpu.VMEM_SHARED((_NP, _D), F32),
    ],
)(_sc_scatter_body)


_BN = 1000
_BE = 2000


def _prep_body(h_ref, wd_ref, ws_ref, b1_ref, a_ref, b_ref):
    h = h_ref[...]
    a_ref[...] = _dot(h, wd_ref[...]) + b1_ref[...]
    b_ref[...] = _dot(h, ws_ref[...])


def _tc_prep(h, wd, ws, b1):
    return pl.pallas_call(
        _prep_body,
        grid=(_N // _BN,),
        in_specs=[
            pl.BlockSpec((_BN, _D), lambda i: (i, 0)),
            pl.BlockSpec((_D, _D), lambda i: (0, 0)),
            pl.BlockSpec((_D, _D), lambda i: (0, 0)),
            pl.BlockSpec((1, _D), lambda i: (0, 0)),
        ],
        out_specs=[
            pl.BlockSpec((_BN, _D), lambda i: (i, 0)),
            pl.BlockSpec((_BN, _D), lambda i: (i, 0)),
        ],
        out_shape=[
            jax.ShapeDtypeStruct((_N, _D), F32),
            jax.ShapeDtypeStruct((_N, _D), F32),
        ],
    )(h, wd, ws, b1.reshape(1, _D))


def _edge_body(g_ref, w_ref, b_ref, o_ref):
    m1 = _silu(g_ref[...])
    o_ref[...] = _silu(_dot(m1, w_ref[...]) + b_ref[...])


def _tc_edge(g, w2, b2):
    return pl.pallas_call(
        _edge_body,
        grid=(_E // _BE,),
        in_specs=[
            pl.BlockSpec((_BE, _D), lambda i: (i, 0)),
            pl.BlockSpec((_D, _D), lambda i: (0, 0)),
            pl.BlockSpec((1, _D), lambda i: (0, 0)),
        ],
        out_specs=pl.BlockSpec((_BE, _D), lambda i: (i, 0)),
        out_shape=jax.ShapeDtypeStruct((_E, _D), F32),
    )(g, w2, b2.reshape(1, _D))


def _node_body(h_ref, a0_ref, a1_ref, w1h_ref, w1a_ref, b1_ref,
               w2_ref, b2_ref, o_ref):
    aggr = a0_ref[0] + a1_ref[0]
    u = _silu(_dot(h_ref[...], w1h_ref[...]) + _dot(aggr, w1a_ref[...])
              + b1_ref[...])
    o_ref[...] = _dot(u, w2_ref[...]) + b2_ref[...]


def _tc_node(h, agg2, w1h, w1a, b1, w2, b2):
    full = lambda i: (0, 0)
    return pl.pallas_call(
        _node_body,
        grid=(_N // _BN,),
        in_specs=[
            pl.BlockSpec((_BN, _D), lambda i: (i, 0)),
            pl.BlockSpec((1, _BN, _D), lambda i: (0, i, 0)),
            pl.BlockSpec((1, _BN, _D), lambda i: (1, i, 0)),
            pl.BlockSpec((_D, _D), full),
            pl.BlockSpec((_D, _D), full),
            pl.BlockSpec((1, _D), full),
            pl.BlockSpec((_D, _D), full),
            pl.BlockSpec((1, _D), full),
        ],
        out_specs=pl.BlockSpec((_BN, _D), lambda i: (i, 0)),
        out_shape=jax.ShapeDtypeStruct((_N, _D), F32),
    )(h, agg2, agg2, w1h, w1a, b1.reshape(1, _D), w2, b2.reshape(1, _D))


def _head_body(h_ref, w11_ref, b11_ref, w12_ref, b12_ref,
               w21_ref, b21_ref, w22_ref, b22_ref, o_ref, acc):
    i = pl.program_id(0)

    @pl.when(i == 0)
    def _():
        acc[...] = jnp.zeros_like(acc)

    h1 = _dot(_silu(_dot(h_ref[...], w11_ref[...]) + b11_ref[...]),
              w12_ref[...]) + b12_ref[...]
    acc[...] += jnp.sum(h1, axis=0, keepdims=True)

    @pl.when(i == pl.num_programs(0) - 1)
    def _():
        pooled = acc[...]
        o_ref[...] = _dot(_silu(_dot(pooled, w21_ref[...]) + b21_ref[...]),
                          w22_ref[...]) + b22_ref[...]


def _tc_head(h, w11, b11, w12, b12, w21, b21, w22, b22):
    nc = w22.shape[1]
    full = lambda i: (0, 0)
    return pl.pallas_call(
        _head_body,
        grid=(_N // _BN,),
        in_specs=[
            pl.BlockSpec((_BN, _D), lambda i: (i, 0)),
            pl.BlockSpec((_D, _D), full),
            pl.BlockSpec((1, _D), full),
            pl.BlockSpec((_D, _D), full),
            pl.BlockSpec((1, _D), full),
            pl.BlockSpec((_D, _D), full),
            pl.BlockSpec((1, _D), full),
            pl.BlockSpec((_D, nc), full),
            pl.BlockSpec((1, nc), full),
        ],
        out_specs=pl.BlockSpec((1, nc), full),
        out_shape=jax.ShapeDtypeStruct((1, nc), F32),
        scratch_shapes=[pltpu.VMEM((1, _D), F32)],
    )(h, w11, b11.reshape(1, _D), w12, b12.reshape(1, _D),
      w21, b21.reshape(1, _D), w22, b22.reshape(1, nc))


def kernel(x, h, edge_index, params):
    xpad = jnp.pad(x, ((0, 0), (0, 1))).reshape(-1)
    eidx = edge_index.reshape(2, _NCH, _C)
    for p in params["layers"]:
        wd = p["e_w1"][:_D]
        ws = p["e_w1"][_D:2 * _D]
        wdist = p["e_w1"][2 * _D]
        a, b = _tc_prep(h, wd, ws, p["e_b1"])
        g = _sc_gather(a, b, xpad, eidx, wdist)
        m2 = _tc_edge(g, p["e_w2"], p["e_b2"])
        agg2 = _sc_scatter(m2, eidx)
        h = _tc_node(h, agg2, p["n_w1"][:_D], p["n_w1"][_D:], p["n_b1"],
                     p["n_w2"], p["n_b2"])
    return _tc_head(h, params["f1_w1"], params["f1_b1"],
                    params["f1_w2"], params["f1_b2"],
                    params["f2_w1"], params["f2_b1"],
                    params["f2_w2"], params["f2_b2"])
